# baseline (speedup 1.0000x reference)
import functools

import jax
import jax.numpy as jnp
from jax.experimental import pallas as pl
from jax.experimental.pallas import tpu as pltpu

_BF = jnp.bfloat16
_DN = ('NHWC', 'HWIO', 'NHWC')


def _vq_body(zp_ref, cb_ref, q_ref):
    zp = zp_ref[...]
    cb = cb_ref[...]
    dots = jax.lax.dot_general(zp, cb, (((1,), (1,)), ((), ())),
                               preferred_element_type=jnp.float32)
    d = (jnp.sum(zp * zp, axis=1, keepdims=True)
         + jnp.sum(cb * cb, axis=1)[None, :]
         - 2.0 * dots)
    idx = jnp.argmin(d, axis=1)
    onehot = (jax.lax.broadcasted_iota(jnp.int32, d.shape, 1)
              == idx[:, None]).astype(jnp.float32)
    q_ref[...] = jnp.dot(onehot, cb, preferred_element_type=jnp.float32)


def _vq_quantize(zp, codebook):
    return pl.pallas_call(
        _vq_body,
        out_shape=jax.ShapeDtypeStruct(zp.shape, jnp.float32),
    )(zp, codebook)


def _deconv_body(x_ref, w_ref, b_ref, o_ref, *, Th, Wp, Co):
    bias = b_ref[...].astype(jnp.float32)
    for p in range(4):
        a, t = p // 2, p % 2
        y = None
        for j in (0, 1):
            for l in (0, 1):
                c = l + t
                r0 = (j + a) * Wp
                slab = x_ref[0, 0, c, r0:r0 + Th * Wp, :]
                wt = w_ref[(2 * j + a) * 4 + (2 * l + t)]
                z = jnp.dot(slab, wt, preferred_element_type=jnp.float32)
                y = z if y is None else y + z
        y = y + bias
        o_ref[0, 0, p] = jnp.maximum(y, 0.2 * y).astype(_BF)


def _deconv_pallas(x, w, b):
    B, H, W, Ci = x.shape
    Co = w.shape[-1]
    S = 1 if H <= 56 else 2
    Th = H // S
    Wp = -(-(W + 2) // 8) * 8
    xw = jnp.pad(x.astype(_BF), ((0, 0), (1, 1), (1, Wp + 1 - W), (0, 0)))
    xs = jnp.stack([
        jnp.stack([xw[:, s * Th:s * Th + Th + 2, c:c + Wp] for c in range(3)],
                  axis=1)
        for s in range(S)], axis=1)
    xs = xs.reshape(B, S, 3, (Th + 2) * Wp, Ci)
    wc = w.astype(_BF).reshape(16, Ci, Co)
    body = functools.partial(_deconv_body, Th=Th, Wp=Wp, Co=Co)
    out = pl.pallas_call(
        body,
        grid=(B, S),
        in_specs=[
            pl.BlockSpec((1, 1, 3, (Th + 2) * Wp, Ci),
                         lambda i, s: (i, s, 0, 0, 0)),
            pl.BlockSpec((16, Ci, Co), lambda i, s: (0, 0, 0)),
            pl.BlockSpec((1, Co), lambda i, s: (0, 0)),
        ],
        out_specs=pl.BlockSpec((1, 1, 4, Th * Wp, Co),
                               lambda i, s: (i, s, 0, 0, 0)),
        out_shape=jax.ShapeDtypeStruct((B, S, 4, Th * Wp, Co), _BF),
    )(xs, wc, b.reshape(1, Co))
    y = out.reshape(B, S, 2, 2, Th, Wp, Co)[:, :, :, :, :, :W]
    y = jnp.transpose(y, (0, 1, 4, 2, 5, 3, 6))
    return y.reshape(B, 2 * H, 2 * W, Co)


def _conv(x, w, b, pad):
    y = jax.lax.conv_general_dilated(x.astype(_BF), w.astype(_BF), (1, 1),
                                     ((pad, pad), (pad, pad)),
                                     dimension_numbers=_DN,
                                     preferred_element_type=jnp.float32)
    return y + b[None, None, None, :]


def _maxpool(x, p):
    return jax.lax.reduce_window(x, -jnp.inf, jax.lax.max, (1, p, p, 1),
                                 (1, p, p, 1), 'VALID')


def _lrelu(x):
    return jax.nn.leaky_relu(x, 0.2)


def kernel(input, enc_params, dec_deconv, dec_conv, codebook):
    pools = [2, 2, 2, 2, 0]
    h = jnp.transpose(input, (0, 2, 3, 1))
    n = len(enc_params)
    for i, (w, b) in enumerate(enc_params):
        k = w.shape[0]
        h = _conv(h, w, b, k // 2)
        if pools[i] > 0:
            h = _maxpool(h, pools[i])
        h = _lrelu(h) if i < n - 1 else jax.nn.sigmoid(h)

    B, H, W, C = h.shape
    zp = h.reshape(-1, C)
    q = _vq_quantize(zp, codebook)
    qz = q.reshape(B, H, W, C)

    for (w, b) in dec_deconv:
        qz = _deconv_pallas(qz, w, b)
    w, b = dec_conv[0]
    qz = _lrelu(_conv(qz, w, b, 1))
    w, b = dec_conv[1]
    qz = jax.nn.sigmoid(_conv(qz, w, b, 0))
    return jnp.transpose(qz, (0, 3, 1, 2))


# speedup vs baseline: 1.4888x; 1.4888x over previous
import jax
import jax.numpy as jnp
from jax.experimental import pallas as pl

_BF = jnp.bfloat16
_DN = ('NHWC', 'HWIO', 'NHWC')


def _vq_body(zp_ref, cb_ref, q_ref):
    zp = zp_ref[...]
    cb = cb_ref[...]
    dots = jax.lax.dot_general(zp, cb, (((1,), (1,)), ((), ())),
                               preferred_element_type=jnp.float32)
    d = (jnp.sum(zp * zp, axis=1, keepdims=True)
         + jnp.sum(cb * cb, axis=1)[None, :]
         - 2.0 * dots)
    idx = jnp.argmin(d, axis=1)
    onehot = (jax.lax.broadcasted_iota(jnp.int32, d.shape, 1)
              == idx[:, None]).astype(jnp.float32)
    q_ref[...] = jnp.dot(onehot, cb, preferred_element_type=jnp.float32)


def _vq_quantize(zp, codebook):
    return pl.pallas_call(
        _vq_body,
        out_shape=jax.ShapeDtypeStruct(zp.shape, jnp.float32),
    )(zp, codebook)


def _conv(x, w, b, pad):
    y = jax.lax.conv_general_dilated(x.astype(_BF), w.astype(_BF), (1, 1),
                                     ((pad, pad), (pad, pad)),
                                     dimension_numbers=_DN,
                                     preferred_element_type=jnp.float32)
    return y + b[None, None, None, :]


def _deconv(x, w, b, k, stride, pad):
    p = k - 1 - pad
    y = jax.lax.conv_general_dilated(x.astype(_BF), w.astype(_BF), (1, 1),
                                     ((p, p), (p, p)),
                                     lhs_dilation=(stride, stride),
                                     dimension_numbers=_DN,
                                     preferred_element_type=jnp.float32)
    return y + b[None, None, None, :]


def _maxpool(x, p):
    return jax.lax.reduce_window(x, -jnp.inf, jax.lax.max, (1, p, p, 1),
                                 (1, p, p, 1), 'VALID')


def _lrelu(x):
    return jax.nn.leaky_relu(x, 0.2)


def kernel(input, enc_params, dec_deconv, dec_conv, codebook):
    pools = [2, 2, 2, 2, 0]
    h = jnp.transpose(input, (0, 2, 3, 1))
    n = len(enc_params)
    for i, (w, b) in enumerate(enc_params):
        k = w.shape[0]
        h = _conv(h, w, b, k // 2)
        if pools[i] > 0:
            h = _maxpool(h, pools[i])
        h = _lrelu(h) if i < n - 1 else jax.nn.sigmoid(h)

    B, H, W, C = h.shape
    zp = h.reshape(-1, C)
    q = _vq_quantize(zp, codebook)
    qz = q.reshape(B, H, W, C)

    for (w, b) in dec_deconv:
        qz = _lrelu(_deconv(qz, w, b, 4, 2, 1))
    w, b = dec_conv[0]
    qz = _lrelu(_conv(qz, w, b, 1))
    w, b = dec_conv[1]
    qz = jax.nn.sigmoid(_conv(qz, w, b, 0))
    return jnp.transpose(qz, (0, 3, 1, 2))
